# baseline (speedup 1.0000x reference)
import functools
import math

import jax
import jax.numpy as jnp
from jax import lax
from jax.experimental import pallas as pl
from jax.experimental.pallas import tpu as pltpu
from jax.experimental.pallas import tpu_sc as plsc

NC, NS, LANES = 2, 16, 16
NW = NC * NS
C = 128
DEGW = 128


def _sc_mesh():
    return plsc.VectorSubcoreMesh(
        core_axis_name="c", subcore_axis_name="s",
        num_cores=NC, num_subcores=NS)


def _make_deg_kernel(n_chunks_per_tile, acc_rows):
    rpt = acc_rows // NS

    @functools.partial(
        pl.kernel,
        out_type=jax.ShapeDtypeStruct((NC, acc_rows, DEGW), jnp.float32),
        mesh=_sc_mesh(),
        scratch_types=[
            pltpu.VMEM((n_chunks_per_tile, C), jnp.int32),
            pltpu.VMEM((C, DEGW), jnp.float32),
            pltpu.VMEM_SHARED((acc_rows, DEGW), jnp.float32),
        ],
    )
    def deg_kernel(dst_hbm, ones_hbm, zdeg_hbm, deg_out, dst_v, ones_v, deg_sh):
        c = lax.axis_index("c")
        s = lax.axis_index("s")
        wid = s * NC + c
        pltpu.sync_copy(zdeg_hbm, deg_sh.at[pl.ds(s * rpt, rpt)])
        pltpu.sync_copy(dst_hbm.at[pl.ds(wid * n_chunks_per_tile,
                                         n_chunks_per_tile)], dst_v)
        pltpu.sync_copy(ones_hbm, ones_v)
        plsc.subcore_barrier()

        def body(j, carry):
            pltpu.sync_copy(ones_v, deg_sh.at[dst_v.at[j]], add=True)
            return carry
        lax.fori_loop(0, n_chunks_per_tile, body, 0)

        plsc.subcore_barrier()
        pltpu.sync_copy(deg_sh.at[pl.ds(s * rpt, rpt)],
                        deg_out.at[c, pl.ds(s * rpt, rpt)])

    return deg_kernel


NBUF = 2
KP = 16
NP0, NP1 = 10, 0


def _make_agg_kernel(acc_rows, d):
    rpt = acc_rows // NS
    k0, k1 = NP0 * KP, NP1 * KP

    @functools.partial(
        pl.kernel,
        out_type=jax.ShapeDtypeStruct((NC, acc_rows, d), jnp.float32),
        mesh=_sc_mesh(),
        scratch_types=[
            pltpu.VMEM((KP, C), jnp.int32),
            pltpu.VMEM((KP, C), jnp.int32),
        ] + [pltpu.VMEM((C, d), jnp.float32)] * NBUF
          + [pltpu.VMEM_SHARED((acc_rows, d), jnp.float32)]
          + [pltpu.SemaphoreType.DMA] * NBUF,
    )
    def agg_kernel(h_hbm, src_hbm, dst_hbm, zrow_hbm, out_hbm,
                   src_v, dst_v, *rest):
        rows_v = rest[:NBUF]
        acc_sh = rest[NBUF]
        sems = rest[NBUF + 1:]
        c = lax.axis_index("c")
        s = lax.axis_index("s")
        pltpu.sync_copy(zrow_hbm, acc_sh.at[pl.ds(s * rpt, rpt)])
        plsc.subcore_barrier()

        tile_base = jnp.where(c == 0, s * k0, NS * k0 + s * k1)
        n_phases = jnp.where(c == 0, NP0, NP1)

        def phase(p, carry):
            base = tile_base + p * KP
            pltpu.sync_copy(src_hbm.at[pl.ds(base, KP)], src_v)
            pltpu.sync_copy(dst_hbm.at[pl.ds(base, KP)], dst_v)
            for bi in range(NBUF):
                pltpu.async_copy(h_hbm.at[src_v.at[bi]], rows_v[bi],
                                 sems[bi])

            def outer(g, carry2):
                for bi in range(NBUF):
                    j = g * NBUF + bi
                    pltpu.make_async_copy(h_hbm.at[src_v.at[j]],
                                          rows_v[bi], sems[bi]).wait()
                    pltpu.sync_copy(rows_v[bi], acc_sh.at[dst_v.at[j]],
                                    add=True)

                    @pl.when(j + NBUF < KP)
                    def _():
                        pltpu.async_copy(h_hbm.at[src_v.at[j + NBUF]],
                                         rows_v[bi], sems[bi])
                return carry2
            lax.fori_loop(0, KP // NBUF, outer, 0)
            return carry
        lax.fori_loop(0, n_phases, phase, 0)

        plsc.subcore_barrier()
        pltpu.sync_copy(acc_sh.at[pl.ds(s * rpt, rpt)],
                        out_hbm.at[c, pl.ds(s * rpt, rpt)])

    return agg_kernel


def _mm_body(x_ref, w_ref, degp_ref, h_ref, dinv_ref):
    deg = (degp_ref[0] + degp_ref[1])[:, :1] + 1.0
    dv = lax.rsqrt(jnp.maximum(deg, 1.0))
    h = jnp.dot(x_ref[...], w_ref[...], preferred_element_type=jnp.float32)
    h_ref[...] = h * dv
    dinv_ref[...] = dv


def _fin_body(parts_ref, h_ref, dinv_ref, b_ref, o_ref):
    o_ref[...] = ((parts_ref[0] + parts_ref[1] + h_ref[...])
                  * dinv_ref[...] + b_ref[...])


def kernel(x, edge_index, W, b):
    n, d = x.shape
    e = edge_index.shape[1]
    assert d % 128 == 0

    tot_rows = NS * (NP0 + NP1) * KP
    ep = tot_rows * C
    assert ep >= e, "edge list larger than the static chunk layout"
    kch = tot_rows // NW
    assert kch % 8 == 0
    src = edge_index[0]
    dst = edge_index[1]
    pad = ep - e
    srcp = jnp.concatenate([src, jnp.zeros((pad,), jnp.int32)]).reshape(ep // C, C)
    dstp = jnp.concatenate([dst, jnp.full((pad,), n, jnp.int32)]).reshape(ep // C, C)

    rpt = ((-(-(n + 1) // NS)) + 7) // 8 * 8
    acc_rows = rpt * NS

    ones_deg = jnp.ones((C, DEGW), jnp.float32)
    zdeg = jnp.zeros((rpt, DEGW), jnp.float32)
    zrow = jnp.zeros((rpt, d), jnp.float32)

    degp = _make_deg_kernel(kch, acc_rows)(dstp, ones_deg, zdeg)

    r = 2000
    grid = n // r
    assert n % r == 0

    h_t, dinv = pl.pallas_call(
        _mm_body,
        grid=(grid,),
        in_specs=[
            pl.BlockSpec((r, d), lambda i: (i, 0)),
            pl.BlockSpec((d, d), lambda i: (0, 0)),
            pl.BlockSpec((NC, r, DEGW), lambda i: (0, i, 0)),
        ],
        out_specs=[
            pl.BlockSpec((r, d), lambda i: (i, 0)),
            pl.BlockSpec((r, 1), lambda i: (i, 0)),
        ],
        out_shape=[
            jax.ShapeDtypeStruct((n, d), jnp.float32),
            jax.ShapeDtypeStruct((n, 1), jnp.float32),
        ],
    )(x, W, degp)

    parts = _make_agg_kernel(acc_rows, d)(h_t, srcp, dstp, zrow)

    out = pl.pallas_call(
        _fin_body,
        grid=(grid,),
        in_specs=[
            pl.BlockSpec((NC, r, d), lambda i: (0, i, 0)),
            pl.BlockSpec((r, d), lambda i: (i, 0)),
            pl.BlockSpec((r, 1), lambda i: (i, 0)),
            pl.BlockSpec((1, d), lambda i: (0, 0)),
        ],
        out_specs=pl.BlockSpec((r, d), lambda i: (i, 0)),
        out_shape=jax.ShapeDtypeStruct((n, d), jnp.float32),
    )(parts, h_t, dinv, b.reshape(1, d))

    return out


# speedup vs baseline: 1.4834x; 1.4834x over previous
import functools
import math

import jax
import jax.numpy as jnp
from jax import lax
from jax.experimental import pallas as pl
from jax.experimental.pallas import tpu as pltpu
from jax.experimental.pallas import tpu_sc as plsc

NC, NS, LANES = 2, 16, 16
NW = NC * NS
C = 128
DEGW = 128


def _sc_mesh():
    return plsc.VectorSubcoreMesh(
        core_axis_name="c", subcore_axis_name="s",
        num_cores=NC, num_subcores=NS)


def _make_deg_kernel(n_chunks_per_tile, acc_rows):
    rpt = acc_rows // NS

    @functools.partial(
        pl.kernel,
        out_type=jax.ShapeDtypeStruct((NC, acc_rows, DEGW), jnp.float32),
        mesh=_sc_mesh(),
        scratch_types=[
            pltpu.VMEM((n_chunks_per_tile, C), jnp.int32),
            pltpu.VMEM((C, DEGW), jnp.float32),
            pltpu.VMEM_SHARED((acc_rows, DEGW), jnp.float32),
        ],
    )
    def deg_kernel(dst_hbm, ones_hbm, zdeg_hbm, deg_out, dst_v, ones_v, deg_sh):
        c = lax.axis_index("c")
        s = lax.axis_index("s")
        wid = s * NC + c
        pltpu.sync_copy(zdeg_hbm, deg_sh.at[pl.ds(s * rpt, rpt)])
        pltpu.sync_copy(dst_hbm.at[pl.ds(wid * n_chunks_per_tile,
                                         n_chunks_per_tile)], dst_v)
        pltpu.sync_copy(ones_hbm, ones_v)
        plsc.subcore_barrier()

        def body(j, carry):
            pltpu.sync_copy(ones_v, deg_sh.at[dst_v.at[j]], add=True)
            return carry
        lax.fori_loop(0, n_chunks_per_tile, body, 0)

        plsc.subcore_barrier()
        pltpu.sync_copy(deg_sh.at[pl.ds(s * rpt, rpt)],
                        deg_out.at[c, pl.ds(s * rpt, rpt)])

    return deg_kernel


NBUF = 2
KP = 8
NP0, NP1 = 19, 1


def _make_agg_kernel(acc_rows, d):
    rpt = acc_rows // NS
    k0, k1 = NP0 * KP, NP1 * KP

    @functools.partial(
        pl.kernel,
        out_type=jax.ShapeDtypeStruct((NC, acc_rows, d), jnp.float32),
        mesh=_sc_mesh(),
        scratch_types=[
            pltpu.VMEM((KP, C), jnp.int32),
            pltpu.VMEM((KP, C), jnp.int32),
        ] + [pltpu.VMEM((C, d), jnp.float32)] * NBUF
          + [pltpu.VMEM_SHARED((acc_rows, d), jnp.float32)]
          + [pltpu.SemaphoreType.DMA] * NBUF,
    )
    def agg_kernel(h_hbm, src_hbm, dst_hbm, zrow_hbm, out_hbm,
                   src_v, dst_v, *rest):
        rows_v = rest[:NBUF]
        acc_sh = rest[NBUF]
        sems = rest[NBUF + 1:]
        c = lax.axis_index("c")
        s = lax.axis_index("s")
        pltpu.sync_copy(zrow_hbm, acc_sh.at[pl.ds(s * rpt, rpt)])
        plsc.subcore_barrier()

        tile_base = jnp.where(c == 0, s * k0, NS * k0 + s * k1)
        n_phases = jnp.where(c == 0, NP0, NP1)

        def phase(p, carry):
            base = tile_base + p * KP
            pltpu.sync_copy(src_hbm.at[pl.ds(base, KP)], src_v)
            pltpu.sync_copy(dst_hbm.at[pl.ds(base, KP)], dst_v)
            for bi in range(NBUF):
                pltpu.async_copy(h_hbm.at[src_v.at[bi]], rows_v[bi],
                                 sems[bi])

            def outer(g, carry2):
                for bi in range(NBUF):
                    j = g * NBUF + bi
                    pltpu.make_async_copy(h_hbm.at[src_v.at[j]],
                                          rows_v[bi], sems[bi]).wait()
                    pltpu.sync_copy(rows_v[bi], acc_sh.at[dst_v.at[j]],
                                    add=True)

                    @pl.when(j + NBUF < KP)
                    def _():
                        pltpu.async_copy(h_hbm.at[src_v.at[j + NBUF]],
                                         rows_v[bi], sems[bi])
                return carry2
            lax.fori_loop(0, KP // NBUF, outer, 0)
            return carry
        lax.fori_loop(0, n_phases, phase, 0)

        plsc.subcore_barrier()
        pltpu.sync_copy(acc_sh.at[pl.ds(s * rpt, rpt)],
                        out_hbm.at[c, pl.ds(s * rpt, rpt)])

    return agg_kernel


def _mm_body(x_ref, w_ref, degp_ref, h_ref, dinv_ref):
    deg = (degp_ref[0] + degp_ref[1])[:, :1] + 1.0
    dv = lax.rsqrt(jnp.maximum(deg, 1.0))
    h = jnp.dot(x_ref[...], w_ref[...], preferred_element_type=jnp.float32)
    h_ref[...] = h * dv
    dinv_ref[...] = dv


def _fin_body(parts_ref, h_ref, dinv_ref, b_ref, o_ref):
    o_ref[...] = ((parts_ref[0] + parts_ref[1] + h_ref[...])
                  * dinv_ref[...] + b_ref[...])


def kernel(x, edge_index, W, b):
    n, d = x.shape
    e = edge_index.shape[1]
    assert d % 128 == 0

    tot_rows = NS * (NP0 + NP1) * KP
    ep = tot_rows * C
    assert ep >= e, "edge list larger than the static chunk layout"
    kch = tot_rows // NW
    assert kch % 8 == 0
    src = edge_index[0]
    dst = edge_index[1]
    pad = ep - e
    srcp = jnp.concatenate([src, jnp.zeros((pad,), jnp.int32)]).reshape(ep // C, C)
    dstp = jnp.concatenate([dst, jnp.full((pad,), n, jnp.int32)]).reshape(ep // C, C)

    rpt = ((-(-(n + 1) // NS)) + 7) // 8 * 8
    acc_rows = rpt * NS

    ones_deg = jnp.ones((C, DEGW), jnp.float32)
    zdeg = jnp.zeros((rpt, DEGW), jnp.float32)
    zrow = jnp.zeros((rpt, d), jnp.float32)

    degp = _make_deg_kernel(kch, acc_rows)(dstp, ones_deg, zdeg)

    r = 2000
    grid = n // r
    assert n % r == 0

    h_t, dinv = pl.pallas_call(
        _mm_body,
        grid=(grid,),
        in_specs=[
            pl.BlockSpec((r, d), lambda i: (i, 0)),
            pl.BlockSpec((d, d), lambda i: (0, 0)),
            pl.BlockSpec((NC, r, DEGW), lambda i: (0, i, 0)),
        ],
        out_specs=[
            pl.BlockSpec((r, d), lambda i: (i, 0)),
            pl.BlockSpec((r, 1), lambda i: (i, 0)),
        ],
        out_shape=[
            jax.ShapeDtypeStruct((n, d), jnp.float32),
            jax.ShapeDtypeStruct((n, 1), jnp.float32),
        ],
    )(x, W, degp)

    parts = _make_agg_kernel(acc_rows, d)(h_t, srcp, dstp, zrow)

    out = pl.pallas_call(
        _fin_body,
        grid=(grid,),
        in_specs=[
            pl.BlockSpec((NC, r, d), lambda i: (0, i, 0)),
            pl.BlockSpec((r, d), lambda i: (i, 0)),
            pl.BlockSpec((r, 1), lambda i: (i, 0)),
            pl.BlockSpec((1, d), lambda i: (0, 0)),
        ],
        out_specs=pl.BlockSpec((r, d), lambda i: (i, 0)),
        out_shape=jax.ShapeDtypeStruct((n, d), jnp.float32),
    )(parts, h_t, dinv, b.reshape(1, d))

    return out
